# baseline (speedup 1.0000x reference)
import functools

import jax
import jax.numpy as jnp
from jax import lax
from jax.experimental import pallas as pl
from jax.experimental.pallas import tpu as pltpu
from jax.experimental.pallas import tpu_sc as plsc

_LN10 = 2.302585092994046

_ROWS = 128
_COLS = 128
_NW = 32
_RPW = _ROWS // _NW
_LANES = 16


def _build():
    mesh = plsc.VectorSubcoreMesh(core_axis_name="c", subcore_axis_name="s")

    @functools.partial(
        pl.kernel,
        mesh=mesh,
        out_type=jax.ShapeDtypeStruct((_ROWS, _COLS), jnp.float32),
        scratch_types=[
            pltpu.VMEM((_RPW, _COLS), jnp.int32),
            pltpu.VMEM((_RPW, _COLS), jnp.float32),
            pltpu.VMEM((_RPW, _COLS), jnp.float32),
            pltpu.VMEM((_RPW, _COLS), jnp.float32),
            pltpu.VMEM((_RPW, _COLS), jnp.float32),
            pltpu.SemaphoreType.DMA,
            pltpu.SemaphoreType.DMA,
            pltpu.SemaphoreType.DMA,
        ],
    )
    def k(ids_hbm, mu_hbm, ls_hbm, nz_hbm, out_hbm,
          idx_v, mu_v, ls_v, nz_v, out_v, gsem, isem, osem):
        wid = lax.axis_index("s") * 2 + lax.axis_index("c")
        base = wid * _RPW
        idx_copies = [
            pltpu.async_copy(ids_hbm.at[base + r], idx_v.at[r], isem)
            for r in range(_RPW)
        ]
        nz_copy = pltpu.async_copy(nz_hbm.at[pl.ds(base, _RPW)], nz_v, isem)
        gathers = []
        for r in range(_RPW):
            idx_copies[r].wait()
            gathers.append(pltpu.async_copy(mu_hbm.at[idx_v.at[r]], mu_v.at[r], gsem))
            gathers.append(pltpu.async_copy(ls_hbm.at[idx_v.at[r]], ls_v.at[r], gsem))
        nz_copy.wait()
        out_copies = []
        for r in range(_RPW):
            gathers[2 * r].wait()
            gathers[2 * r + 1].wait()
            for i in range(_COLS // _LANES):
                sl = pl.ds(i * _LANES, _LANES)
                m = mu_v[r, sl]
                s = ls_v[r, sl]
                z = nz_v[r, sl]
                out_v[r, sl] = jnp.exp((m + jnp.exp(s) * z) * _LN10)
            out_copies.append(
                pltpu.async_copy(out_v.at[r], out_hbm.at[base + r], osem)
            )
        for c in out_copies:
            c.wait()

    return k


_sc_kernel = _build()


def kernel(batch_size, family_ids, mu, log_sigma, noise):
    ids2 = family_ids.astype(jnp.int32).reshape(_ROWS, _COLS)
    nz2 = noise.reshape(_ROWS, _COLS)
    out = _sc_kernel(ids2, mu, log_sigma, nz2)
    return out.reshape(-1)


# speedup vs baseline: 1.0195x; 1.0195x over previous
import functools

import jax
import jax.numpy as jnp
from jax import lax
from jax.experimental import pallas as pl
from jax.experimental.pallas import tpu as pltpu
from jax.experimental.pallas import tpu_sc as plsc

_LN10 = 2.302585092994046

_ROWS = 128
_COLS = 128
_NW = 32
_RPW = _ROWS // _NW
_EPW = _RPW * _COLS
_LANES = 16


def _build():
    mesh = plsc.VectorSubcoreMesh(core_axis_name="c", subcore_axis_name="s")

    @functools.partial(
        pl.kernel,
        mesh=mesh,
        out_type=jax.ShapeDtypeStruct((_ROWS * _COLS,), jnp.float32),
        scratch_types=[
            pltpu.VMEM((_RPW, _COLS), jnp.int32),
            pltpu.VMEM((_EPW,), jnp.float32),
            pltpu.VMEM((_EPW,), jnp.float32),
            pltpu.VMEM((_EPW,), jnp.float32),
            pltpu.VMEM((_EPW,), jnp.float32),
            pltpu.SemaphoreType.DMA,
            pltpu.SemaphoreType.DMA,
        ],
    )
    def k(ids_hbm, mu_hbm, ls_hbm, nz_hbm, out_hbm,
          idx_v, mu_v, ls_v, nz_v, out_v, gsem, isem):
        wid = lax.axis_index("s") * 2 + lax.axis_index("c")
        rbase = wid * _RPW
        ebase = wid * _EPW
        pltpu.sync_copy(ids_hbm.at[pl.ds(rbase, _RPW)], idx_v)
        gathers = []
        for r in range(_RPW):
            gathers.append(pltpu.async_copy(
                mu_hbm.at[idx_v.at[r]], mu_v.at[pl.ds(r * _COLS, _COLS)], gsem))
            gathers.append(pltpu.async_copy(
                ls_hbm.at[idx_v.at[r]], ls_v.at[pl.ds(r * _COLS, _COLS)], gsem))
        nz_copy = pltpu.async_copy(nz_hbm.at[pl.ds(ebase, _EPW)], nz_v, isem)
        nz_copy.wait()
        for c in gathers:
            c.wait()

        def body(i, _):
            sl = pl.ds(pl.multiple_of(i * _LANES, _LANES), _LANES)
            out_v[sl] = jnp.exp((mu_v[sl] + jnp.exp(ls_v[sl]) * nz_v[sl]) * _LN10)
            return _

        lax.fori_loop(0, _EPW // _LANES, body, 0, unroll=4)
        pltpu.sync_copy(out_v, out_hbm.at[pl.ds(ebase, _EPW)])

    return k


_sc_kernel = _build()


def kernel(batch_size, family_ids, mu, log_sigma, noise):
    ids2 = family_ids.astype(jnp.int32).reshape(_ROWS, _COLS)
    out = _sc_kernel(ids2, mu, log_sigma, noise)
    return out
